# Initial kernel scaffold; baseline (speedup 1.0000x reference)
import jax
import jax.numpy as jnp
from jax.experimental import pallas as pl


def kernel(embed_init, triplets, weights):
    raise NotImplementedError("write your pallas kernel here")


# speedup vs baseline: 2.1703x; 2.1703x over previous
import functools

import jax
import jax.numpy as jnp
from jax import lax
from jax.experimental import pallas as pl
from jax.experimental.pallas import tpu as pltpu
from jax.experimental.pallas import tpu_sc as plsc

NC = 2
NS = 16
NW = NC * NS
L = 16
CHUNK = 2000


def _unpack(p):
    x = plsc.bitcast(lax.shift_left(p, 16), jnp.float32)
    y = plsc.bitcast(lax.bitwise_and(p, jnp.int32(-65536)), jnp.float32)
    return x, y


def _sc_body(n_chunks, trip_hbm, w_hbm, table_hbm, loss_out, viol_out,
             table_v, trip_v, w_v, stage_v):
    c = lax.axis_index("c")
    s = lax.axis_index("s")
    wid = s * NC + c

    pltpu.sync_copy(table_hbm, table_v)

    iota = lax.iota(jnp.int32, L)
    iota3 = iota * 3
    zero = jnp.zeros((L,), jnp.float32)

    def vec_body(v, carry):
        lv, vv = carry
        base = v * L
        pos = iota3 + base * 3
        ii = plsc.load_gather(trip_v, [pos])
        jj = plsc.load_gather(trip_v, [pos + 1])
        kk = plsc.load_gather(trip_v, [pos + 2])
        xi, yi = _unpack(plsc.load_gather(table_v, [ii]))
        xj, yj = _unpack(plsc.load_gather(table_v, [jj]))
        xk, yk = _unpack(plsc.load_gather(table_v, [kk]))
        dxij = xi - xj
        dyij = yi - yj
        dxik = xi - xk
        dyik = yi - yk
        dij = 1.0 + dxij * dxij + dyij * dyij
        dik = 1.0 + dxik * dxik + dyik * dyik
        w = w_v[pl.ds(base, L)]
        lv = lv + w * (dij / (dij + dik))
        vv = vv + jnp.where(dij > dik, 1.0, 0.0).astype(jnp.float32)
        return lv, vv

    def chunk_body(t, carry):
        g = wid + t * NW
        pltpu.sync_copy(trip_hbm.at[pl.ds(g * (3 * CHUNK), 3 * CHUNK)], trip_v)
        pltpu.sync_copy(w_hbm.at[pl.ds(g * CHUNK, CHUNK)], w_v)
        return lax.fori_loop(0, CHUNK // L, vec_body, carry)

    n_mine = (n_chunks - wid + NW - 1) // NW
    lv, vv = lax.fori_loop(0, n_mine, chunk_body, (zero, zero))

    stage_v[...] = lv
    pltpu.sync_copy(stage_v, loss_out.at[wid])
    stage_v[...] = vv
    pltpu.sync_copy(stage_v, viol_out.at[wid])


def kernel(embed_init, triplets, weights):
    n = embed_init.shape[0]
    T = triplets.shape[0]

    b16 = lax.bitcast_convert_type(embed_init.astype(jnp.bfloat16), jnp.uint16)
    b32 = b16.astype(jnp.uint32)
    packed = lax.bitcast_convert_type(b32[:, 0] | (b32[:, 1] << 16), jnp.int32)

    pad = (-T) % CHUNK
    trips = triplets.astype(jnp.int32)
    w = weights.astype(jnp.float32)
    if pad:
        trips = jnp.concatenate([trips, jnp.zeros((pad, 3), jnp.int32)])
        w = jnp.concatenate([w, jnp.zeros((pad,), jnp.float32)])
    n_chunks = (T + pad) // CHUNK
    trip_flat = trips.reshape(-1)

    mesh = plsc.VectorSubcoreMesh(
        core_axis_name="c", subcore_axis_name="s", num_cores=NC, num_subcores=NS
    )
    fn = pl.kernel(
        functools.partial(_sc_body, n_chunks),
        out_type=(
            jax.ShapeDtypeStruct((NW, L), jnp.float32),
            jax.ShapeDtypeStruct((NW, L), jnp.float32),
        ),
        mesh=mesh,
        scratch_types=[
            pltpu.VMEM((n,), jnp.int32),
            pltpu.VMEM((3 * CHUNK,), jnp.int32),
            pltpu.VMEM((CHUNK,), jnp.float32),
            pltpu.VMEM((L,), jnp.float32),
        ],
        compiler_params=pltpu.CompilerParams(needs_layout_passes=False),
    )
    loss_parts, viol_parts = fn(trip_flat, w, packed)
    return jnp.sum(loss_parts), jnp.sum(viol_parts)


# speedup vs baseline: 51.6186x; 23.7837x over previous
import functools

import jax
import jax.numpy as jnp
from jax import lax
from jax.experimental import pallas as pl
from jax.experimental.pallas import tpu as pltpu
from jax.experimental.pallas import tpu_sc as plsc

NC = 2
NS = 16
NW = NC * NS
L = 16
CHUNK = 2000


def _unpack(p):
    x = plsc.bitcast(lax.shift_left(p, 16), jnp.float32)
    y = plsc.bitcast(lax.bitwise_and(p, jnp.int32(-65536)), jnp.float32)
    return x, y


def _sc_body(n_chunks, i_hbm, j_hbm, k_hbm, w_hbm, table_hbm,
             loss_out, viol_out, table_v, i_v, j_v, k_v, w_v, stage_v):
    c = lax.axis_index("c")
    s = lax.axis_index("s")
    wid = s * NC + c

    pltpu.sync_copy(table_hbm, table_v)

    zero = jnp.zeros((L,), jnp.float32)

    def vec_body(v, carry):
        lv, vv = carry
        base = v * L
        sl = pl.ds(base, L)
        xi, yi = _unpack(plsc.load_gather(table_v, [i_v[sl]]))
        xj, yj = _unpack(plsc.load_gather(table_v, [j_v[sl]]))
        xk, yk = _unpack(plsc.load_gather(table_v, [k_v[sl]]))
        dxij = xi - xj
        dyij = yi - yj
        dxik = xi - xk
        dyik = yi - yk
        dij = 1.0 + dxij * dxij + dyij * dyij
        dik = 1.0 + dxik * dxik + dyik * dyik
        w = w_v[sl]
        lv = lv + w * (dij / (dij + dik))
        vv = vv + jnp.where(dij > dik, 1.0, 0.0).astype(jnp.float32)
        return lv, vv

    def chunk_body(t, carry):
        g = wid + t * NW
        sl = pl.ds(g * CHUNK, CHUNK)
        pltpu.sync_copy(i_hbm.at[sl], i_v)
        pltpu.sync_copy(j_hbm.at[sl], j_v)
        pltpu.sync_copy(k_hbm.at[sl], k_v)
        pltpu.sync_copy(w_hbm.at[sl], w_v)
        return lax.fori_loop(0, CHUNK // L, vec_body, carry)

    n_mine = (n_chunks - wid + NW - 1) // NW
    lv, vv = lax.fori_loop(0, n_mine, chunk_body, (zero, zero))

    stage_v[...] = lv
    pltpu.sync_copy(stage_v, loss_out.at[wid])
    stage_v[...] = vv
    pltpu.sync_copy(stage_v, viol_out.at[wid])


def kernel(embed_init, triplets, weights):
    n = embed_init.shape[0]
    T = triplets.shape[0]

    b16 = lax.bitcast_convert_type(embed_init.astype(jnp.bfloat16), jnp.uint16)
    b32 = b16.astype(jnp.uint32)
    packed = lax.bitcast_convert_type(b32[:, 0] | (b32[:, 1] << 16), jnp.int32)

    trips = triplets.astype(jnp.int32)
    w = weights.astype(jnp.float32)
    pad = (-T) % CHUNK
    if pad:
        trips = jnp.concatenate([trips, jnp.zeros((pad, 3), jnp.int32)])
        w = jnp.concatenate([w, jnp.zeros((pad,), jnp.float32)])
    n_chunks = (T + pad) // CHUNK
    ti, tj, tk = trips[:, 0], trips[:, 1], trips[:, 2]

    mesh = plsc.VectorSubcoreMesh(
        core_axis_name="c", subcore_axis_name="s", num_cores=NC, num_subcores=NS
    )
    fn = pl.kernel(
        functools.partial(_sc_body, n_chunks),
        out_type=(
            jax.ShapeDtypeStruct((NW, L), jnp.float32),
            jax.ShapeDtypeStruct((NW, L), jnp.float32),
        ),
        mesh=mesh,
        scratch_types=[
            pltpu.VMEM((n,), jnp.int32),
            pltpu.VMEM((CHUNK,), jnp.int32),
            pltpu.VMEM((CHUNK,), jnp.int32),
            pltpu.VMEM((CHUNK,), jnp.int32),
            pltpu.VMEM((CHUNK,), jnp.float32),
            pltpu.VMEM((L,), jnp.float32),
        ],
        compiler_params=pltpu.CompilerParams(needs_layout_passes=False),
    )
    loss_parts, viol_parts = fn(ti, tj, tk, w, packed)
    return jnp.sum(loss_parts), jnp.sum(viol_parts)
